# baseline (speedup 1.0000x reference)
import functools

import jax
import jax.numpy as jnp
from jax import lax
from jax.experimental import pallas as pl
from jax.experimental.pallas import tpu as pltpu
from jax.experimental.pallas import tpu_sc as plsc

_B = 16384
_D = 64
_L = 16
_NW = 32
_BPW = _B // _NW
_G = _BPW // _L

_mesh = plsc.VectorSubcoreMesh(core_axis_name="c", subcore_axis_name="s")


@functools.partial(
    pl.kernel,
    mesh=_mesh,
    out_type=(
        jax.ShapeDtypeStruct((_B,), jnp.float32),
        jax.ShapeDtypeStruct((_NW, _L), jnp.float32),
    ),
    scratch_types=[
        pltpu.VMEM((_BPW,), jnp.int32),
        pltpu.VMEM((_BPW,), jnp.int32),
        pltpu.VMEM((_BPW,), jnp.float32),
        pltpu.VMEM((_BPW, _D), jnp.float32),
        pltpu.VMEM((_BPW, _D), jnp.float32),
        pltpu.VMEM((_BPW,), jnp.float32),
        pltpu.VMEM((_BPW,), jnp.float32),
        pltpu.VMEM((_BPW,), jnp.float32),
        pltpu.VMEM((_L * _L,), jnp.float32),
        pltpu.VMEM((_L * _L,), jnp.float32),
        pltpu.VMEM((_L * _L,), jnp.float32),
        pltpu.VMEM((_L,), jnp.float32),
        pltpu.VMEM((_L,), jnp.float32),
        pltpu.SemaphoreType.DMA,
    ],
    compiler_params=pltpu.CompilerParams(needs_layout_passes=False,
                                         use_tc_tiling_on_sc=False),
)
def _mf_sc(du_hbm, di_hbm, dr_hbm, uw_hbm, iw_hbm, ub_hbm, ib_hbm, bias_hbm,
           pred_hbm, part_hbm,
           uidx_v, iidx_v, rat_v, urows_v, irows_v, ubr_v, ibr_v,
           pred_v, sbufS, sbufU, sbufI, acc_v, bias_v, sem):
    wid = lax.axis_index("s") * 2 + lax.axis_index("c")
    base = wid * _BPW

    pltpu.sync_copy(du_hbm.at[pl.ds(base, _BPW)], uidx_v)
    pltpu.sync_copy(di_hbm.at[pl.ds(base, _BPW)], iidx_v)
    c1 = pltpu.async_copy(uw_hbm.at[uidx_v], urows_v, sem)
    c2 = pltpu.async_copy(iw_hbm.at[iidx_v], irows_v, sem)
    c3 = pltpu.async_copy(ub_hbm.at[uidx_v], ubr_v, sem)
    c4 = pltpu.async_copy(ib_hbm.at[iidx_v], ibr_v, sem)
    pltpu.sync_copy(dr_hbm.at[pl.ds(base, _BPW)], rat_v)
    pltpu.sync_copy(bias_hbm, bias_v)
    c1.wait()
    c2.wait()
    c3.wait()
    c4.wait()

    bias_vec = bias_v[...]
    iota = lax.iota(jnp.int32, _L)

    def group(g, acc):
        for r16 in range(_L):
            row = g * _L + r16
            s = pu = pi = None
            for k in range(_D // _L):
                u = urows_v[row, pl.ds(k * _L, _L)]
                i = irows_v[row, pl.ds(k * _L, _L)]
                if s is None:
                    s, pu, pi = u * i, u, i
                else:
                    s, pu, pi = s + u * i, pu + u, pi + i
            sbufS[pl.ds(r16 * _L, _L)] = s
            sbufU[pl.ds(r16 * _L, _L)] = pu
            sbufI[pl.ds(r16 * _L, _L)] = pi
        dot = su = si = None
        for l in range(_L):
            flat = iota * _L + l
            ts = plsc.load_gather(sbufS, [flat])
            tu = plsc.load_gather(sbufU, [flat])
            ti = plsc.load_gather(sbufI, [flat])
            if dot is None:
                dot, su, si = ts, tu, ti
            else:
                dot, su, si = dot + ts, su + tu, si + ti
        ub_g = ubr_v[pl.ds(g * _L, _L)]
        ib_g = ibr_v[pl.ds(g * _L, _L)]
        pred_g = dot + ub_g * si + ib_g * su + (ub_g * ib_g) * float(_D) + bias_vec
        pred_v[pl.ds(g * _L, _L)] = pred_g
        err = pred_g - rat_v[pl.ds(g * _L, _L)]
        return acc + err * err

    acc = lax.fori_loop(0, _G, group, jnp.zeros((_L,), jnp.float32))
    acc_v[...] = acc

    pltpu.sync_copy(pred_v, pred_hbm.at[pl.ds(base, _BPW)])
    pltpu.sync_copy(acc_v, part_hbm.at[wid])


def _loss_body(part_ref, o_ref):
    o_ref[...] = jnp.sum(part_ref[...]).reshape(1, 1) * (1.0 / _B)


@jax.jit
def kernel(data_user, data_item, data_rating, user_weight, item_weight,
           user_bias, item_bias, bias):
    ub_flat = user_bias.reshape(-1)
    ib_flat = item_bias.reshape(-1)
    bias16 = jnp.broadcast_to(bias, (_L,))
    pred, partials = _mf_sc(data_user, data_item, data_rating,
                            user_weight, item_weight,
                            ub_flat, ib_flat, bias16)
    loss2 = pl.pallas_call(
        _loss_body,
        out_shape=jax.ShapeDtypeStruct((1, 1), jnp.float32),
    )(partials)
    return pred, loss2[0, 0]
